# baseline (speedup 1.0000x reference)
import jax
import jax.numpy as jnp
from jax.experimental import pallas as pl
from jax.experimental.pallas import tpu as pltpu

N = 4096
D = 128
TILE = 128
G = N // TILE
PACK = 32
NW = N // PACK
NEG = float("-inf")


def _dott(a, b):
    return jax.lax.dot_general(a.astype(jnp.bfloat16), b.astype(jnp.bfloat16),
                               (((1,), (1,)), ((), ())),
                               preferred_element_type=jnp.float32)


def _matvec(m, v):
    return jax.lax.dot_general(m, v, (((1,), (0,)), ((), ())),
                               preferred_element_type=jnp.float32)


def _bf(a):
    return a.astype(jnp.bfloat16).astype(jnp.float32)


def _body(adj12_ref, adj21_ref, f1_ref, f2_ref, x1_ref, x2_ref,
          w1s0_ref, w1n0_ref, w2s0_ref, w2n0_ref,
          w1s1_ref, w1n1_ref, w2s1_ref, w2n1_ref,
          o1_ref, o2_ref,
          mbf_scr, b21_scr, g1c_scr, g2r_scr, s1c_scr, s2c_scr):
    i = pl.program_id(0)
    t = jax.lax.rem(i, G)
    rows = pl.ds(t * TILE, TILE)

    @pl.when(i < G)
    def phase1():
        adj12 = adj12_ref[...]
        adj21 = adj21_ref[...]

        s2 = jnp.sum(jnp.where(adj12 > 0, f1_ref[...], 0.0),
                     axis=1, keepdims=True)
        s2c_scr[rows, :] = s2
        mbf_scr[rows, :] = adj12.astype(jnp.bfloat16)

        m21 = adj21 > 0
        mx0 = jnp.max(jnp.where(m21, f2_ref[...], NEG), axis=1, keepdims=True)
        s1 = jnp.where(mx0 == NEG, 0.0, mx0)
        s1c_scr[rows, :] = s1
        au = adj21.astype(jnp.uint32)
        b = jnp.zeros((TILE, NW), dtype=jnp.uint32)
        for k in range(PACK):
            b = b + (au[:, k * NW:(k + 1) * NW] << k)
        b21_scr[rows, :] = b

        c1 = jnp.sum(_bf(w1n0_ref[0, :]))
        c2 = jnp.sum(_bf(w2n0_ref[0, :]))
        a1 = jnp.sum(_bf(x1_ref[...]) * _bf(w1s0_ref[0:1, :]), axis=1, keepdims=True)
        a2 = jnp.sum(_bf(x2_ref[...]) * _bf(w2s0_ref[0:1, :]), axis=1, keepdims=True)
        g1c_scr[rows, :] = jnp.maximum(a1 + _bf(s1) * c1, 0.0)
        g2 = jnp.maximum(a2 + _bf(s2) * c2, 0.0)
        g2r_scr[0:1, rows] = g2.reshape(1, TILE)

    @pl.when(i >= G)
    def phase2():
        b21 = b21_scr[rows, :]
        mx = jnp.zeros((TILE, NW), dtype=jnp.float32)
        for k in range(PACK):
            mk = ((b21 >> k) & 1) > 0
            g2k = g2r_scr[0:1, k * NW:(k + 1) * NW]
            mx = jnp.maximum(mx, jnp.where(mk, g2k, 0.0))
        s1p = jnp.max(mx, axis=1, keepdims=True)

        g1 = g1c_scr[...]
        hi = g1.astype(jnp.bfloat16)
        lo = (g1 - hi.astype(jnp.float32)).astype(jnp.bfloat16)
        m = mbf_scr[rows, :]
        s2p = _matvec(m, hi) + _matvec(m, lo)

        s1 = s1c_scr[rows, :]
        s2 = s2c_scr[rows, :]
        r1n0 = jnp.sum(_bf(w1n0_ref[...]), axis=1)[None, :]
        r2n0 = jnp.sum(_bf(w2n0_ref[...]), axis=1)[None, :]
        r1n1 = jnp.sum(_bf(w1n1_ref[...]), axis=1)[None, :]
        r2n1 = jnp.sum(_bf(w2n1_ref[...]), axis=1)[None, :]
        h1 = jnp.maximum(_dott(x1_ref[...], w1s0_ref[...]) + _bf(s1) * r1n0, 0.0)
        h2 = jnp.maximum(_dott(x2_ref[...], w2s0_ref[...]) + _bf(s2) * r2n0, 0.0)
        o1_ref[...] = _dott(h1, w1s1_ref[...]) + _bf(s1p) * r1n1
        o2_ref[...] = _dott(h2, w2s1_ref[...]) + _bf(s2p) * r2n1


def kernel(x1, x2, adj_1to2, adj_2to1,
           l0_w1_self, l0_w1_neigh, l0_w2_self, l0_w2_neigh,
           l1_w1_self, l1_w1_neigh, l1_w2_self, l1_w2_neigh):
    f1 = x1[:, 0].reshape(1, N)
    f2 = x2[:, 0].reshape(1, N)

    adj_t = lambda i: (jnp.minimum(i, G - 1), 0)
    mod_t = lambda i: (jax.lax.rem(i, G), 0)
    out_t = lambda i: (jnp.maximum(i - G, 0), 0)
    full = lambda i: (0, 0)

    o1, o2 = pl.pallas_call(
        _body,
        grid=(2 * G,),
        in_specs=[
            pl.BlockSpec((TILE, N), adj_t),
            pl.BlockSpec((TILE, N), adj_t),
            pl.BlockSpec((1, N), full),
            pl.BlockSpec((1, N), full),
            pl.BlockSpec((TILE, D), mod_t),
            pl.BlockSpec((TILE, D), mod_t),
            pl.BlockSpec((D, D), full),
            pl.BlockSpec((D, D), full),
            pl.BlockSpec((D, D), full),
            pl.BlockSpec((D, D), full),
            pl.BlockSpec((D, D), full),
            pl.BlockSpec((D, D), full),
            pl.BlockSpec((D, D), full),
            pl.BlockSpec((D, D), full),
        ],
        out_specs=[
            pl.BlockSpec((TILE, D), out_t),
            pl.BlockSpec((TILE, D), out_t),
        ],
        out_shape=[
            jax.ShapeDtypeStruct((N, D), jnp.float32),
            jax.ShapeDtypeStruct((N, D), jnp.float32),
        ],
        scratch_shapes=[
            pltpu.VMEM((N, N), jnp.bfloat16),
            pltpu.VMEM((N, NW), jnp.uint32),
            pltpu.VMEM((N, 1), jnp.float32),
            pltpu.VMEM((1, N), jnp.float32),
            pltpu.VMEM((N, 1), jnp.float32),
            pltpu.VMEM((N, 1), jnp.float32),
        ],
        compiler_params=pltpu.CompilerParams(
            dimension_semantics=("arbitrary",)),
    )(adj_1to2, adj_2to1, f1, f2, x1, x2,
      l0_w1_self, l0_w1_neigh, l0_w2_self, l0_w2_neigh,
      l1_w1_self, l1_w1_neigh, l1_w2_self, l1_w2_neigh)

    return (o1, o2)


# speedup vs baseline: 1.2915x; 1.2915x over previous
import jax
import jax.numpy as jnp
from jax.experimental import pallas as pl
from jax.experimental.pallas import tpu as pltpu

N = 4096
D = 128
T = 256
P = N // T // 2
HT = N // 2
PACK = 32
NW = N // PACK
NEG = float("-inf")


def _dott(a, b):
    return jax.lax.dot_general(a.astype(jnp.bfloat16), b.astype(jnp.bfloat16),
                               (((1,), (1,)), ((), ())),
                               preferred_element_type=jnp.float32)


def _matmat(m, v):
    return jax.lax.dot_general(m, v, (((1,), (0,)), ((), ())),
                               preferred_element_type=jnp.float32)


def _bf(a):
    return a.astype(jnp.bfloat16).astype(jnp.float32)


def _body(a12a_ref, a12b_ref, a21c_ref, a21d_ref, f1_ref, f2_ref,
          x1c_ref, x1d_ref, x2a_ref, x2b_ref,
          w1s0_ref, w1n0_ref, w2s0_ref, w2n0_ref,
          w1s1_ref, w1n1_ref, w2s1_ref, w2n1_ref,
          o1a_ref, o1b_ref, o2_ref,
          b12_scr, h2_scr, g2r_scr, s2p_scr):
    i = pl.program_id(0)

    @pl.when(i < P)
    def phase_a():
        j = i
        for half, (aref, xref) in enumerate(((a12a_ref, x2a_ref),
                                             (a12b_ref, x2b_ref))):
            rows = pl.ds(half * HT + j * T, T)
            adj = aref[...]
            s2 = jnp.sum(jnp.where(adj > 0, f1_ref[...], 0.0),
                         axis=1, keepdims=True)
            au = adj.astype(jnp.uint32)
            b = jnp.zeros((T, NW), dtype=jnp.uint32)
            for k in range(PACK):
                b = b + (au[:, k * NW:(k + 1) * NW] << k)
            b12_scr[rows, :] = b

            c2 = jnp.sum(_bf(w2n0_ref[0, :]))
            a2 = jnp.sum(_bf(xref[...]) * _bf(w2s0_ref[0:1, :]),
                         axis=1, keepdims=True)
            g2 = jnp.maximum(a2 + _bf(s2) * c2, 0.0)
            g2r_scr[0:1, rows] = g2.reshape(1, T)

            r2n0 = jnp.sum(_bf(w2n0_ref[...]), axis=1)[None, :]
            h2 = jnp.maximum(_dott(xref[...], w2s0_ref[...]) + _bf(s2) * r2n0,
                             0.0)
            h2_scr[rows, :] = h2

    @pl.when((i >= P) & (i < 2 * P))
    def phase_b():
        j = i - P
        acc = jnp.zeros((N, 1), dtype=jnp.float32)
        for half, (aref, xref, oref) in enumerate(((a21c_ref, x1c_ref, o1a_ref),
                                                   (a21d_ref, x1d_ref, o1b_ref))):
            adj = aref[...]
            m = adj > 0
            mx0 = jnp.max(jnp.where(m, f2_ref[...], NEG), axis=1, keepdims=True)
            s1 = jnp.where(mx0 == NEG, 0.0, mx0)
            s1p = jnp.max(jnp.where(m, g2r_scr[...], 0.0),
                          axis=1, keepdims=True)

            c1 = jnp.sum(_bf(w1n0_ref[0, :]))
            a1 = jnp.sum(_bf(xref[...]) * _bf(w1s0_ref[0:1, :]),
                         axis=1, keepdims=True)
            g1 = jnp.maximum(a1 + _bf(s1) * c1, 0.0)

            r1n0 = jnp.sum(_bf(w1n0_ref[...]), axis=1)[None, :]
            r1n1 = jnp.sum(_bf(w1n1_ref[...]), axis=1)[None, :]
            h1 = jnp.maximum(_dott(xref[...], w1s0_ref[...]) + _bf(s1) * r1n0,
                             0.0)
            oref[...] = _dott(h1, w1s1_ref[...]) + _bf(s1p) * r1n1

            hi = g1.astype(jnp.bfloat16)
            lo = (g1 - hi.astype(jnp.float32)).astype(jnp.bfloat16)
            hl = jnp.concatenate([hi, lo], axis=1)
            base = 2 * (half * P + j)
            bits = b12_scr[...]
            for q in range(2):
                k = base + q
                mk = ((bits >> k) & jnp.uint32(1)).astype(jnp.bfloat16)
                seg = hl[q * NW:(q + 1) * NW, :]
                acc = acc + jnp.sum(_matmat(mk, seg), axis=1, keepdims=True)
        prev = jnp.where(j == 0, jnp.zeros((N, 1), jnp.float32), s2p_scr[...])
        s2p_scr[...] = prev + acc

    @pl.when(i >= 2 * P)
    def phase_c():
        j = i - 2 * P
        rows = pl.ds(j * (2 * T), 2 * T)
        h2 = h2_scr[rows, :]
        s2p = s2p_scr[rows, :]
        r2n1 = jnp.sum(_bf(w2n1_ref[...]), axis=1)[None, :]
        o2_ref[...] = _dott(h2, w2s1_ref[...]) + _bf(s2p) * r2n1


def kernel(x1, x2, adj_1to2, adj_2to1,
           l0_w1_self, l0_w1_neigh, l0_w2_self, l0_w2_neigh,
           l1_w1_self, l1_w1_neigh, l1_w2_self, l1_w2_neigh):
    f1 = x1[:, 0].reshape(1, N)
    f2 = x2[:, 0].reshape(1, N)

    top_a = lambda i: (jnp.minimum(i, P - 1), 0)
    bot_a = lambda i: (P + jnp.minimum(i, P - 1), 0)
    top_b = lambda i: (jnp.clip(i - P, 0, P - 1), 0)
    bot_b = lambda i: (P + jnp.clip(i - P, 0, P - 1), 0)
    out_c = lambda i: (jnp.clip(i - 2 * P, 0, P - 1), 0)
    full = lambda i: (0, 0)

    o1a, o1b, o2 = pl.pallas_call(
        _body,
        grid=(3 * P,),
        in_specs=[
            pl.BlockSpec((T, N), top_a),
            pl.BlockSpec((T, N), bot_a),
            pl.BlockSpec((T, N), top_b),
            pl.BlockSpec((T, N), bot_b),
            pl.BlockSpec((1, N), full),
            pl.BlockSpec((1, N), full),
            pl.BlockSpec((T, D), top_b),
            pl.BlockSpec((T, D), bot_b),
            pl.BlockSpec((T, D), top_a),
            pl.BlockSpec((T, D), bot_a),
            pl.BlockSpec((D, D), full),
            pl.BlockSpec((D, D), full),
            pl.BlockSpec((D, D), full),
            pl.BlockSpec((D, D), full),
            pl.BlockSpec((D, D), full),
            pl.BlockSpec((D, D), full),
            pl.BlockSpec((D, D), full),
            pl.BlockSpec((D, D), full),
        ],
        out_specs=[
            pl.BlockSpec((T, D), top_b),
            pl.BlockSpec((T, D), top_b),
            pl.BlockSpec((2 * T, D), out_c),
        ],
        out_shape=[
            jax.ShapeDtypeStruct((HT, D), jnp.float32),
            jax.ShapeDtypeStruct((HT, D), jnp.float32),
            jax.ShapeDtypeStruct((N, D), jnp.float32),
        ],
        scratch_shapes=[
            pltpu.VMEM((N, NW), jnp.uint32),
            pltpu.VMEM((N, D), jnp.float32),
            pltpu.VMEM((1, N), jnp.float32),
            pltpu.VMEM((N, 1), jnp.float32),
        ],
        compiler_params=pltpu.CompilerParams(
            dimension_semantics=("arbitrary",)),
    )(adj_1to2, adj_1to2, adj_2to1, adj_2to1, f1, f2, x1, x1, x2, x2,
      l0_w1_self, l0_w1_neigh, l0_w2_self, l0_w2_neigh,
      l1_w1_self, l1_w1_neigh, l1_w2_self, l1_w2_neigh)

    return (jnp.concatenate([o1a, o1b], axis=0), o2)


# speedup vs baseline: 1.3012x; 1.0075x over previous
import jax
import jax.numpy as jnp
from jax.experimental import pallas as pl
from jax.experimental.pallas import tpu as pltpu

N = 4096
D = 128
T = 512
P = N // T // 2
HT = N // 2
PACK = 32
NW = N // PACK
QT = T // NW
NEG = float("-inf")


def _dott(a, b):
    return jax.lax.dot_general(a.astype(jnp.bfloat16), b.astype(jnp.bfloat16),
                               (((1,), (1,)), ((), ())),
                               preferred_element_type=jnp.float32)


def _matmat(m, v):
    return jax.lax.dot_general(m, v, (((1,), (0,)), ((), ())),
                               preferred_element_type=jnp.float32)


def _bf(a):
    return a.astype(jnp.bfloat16).astype(jnp.float32)


def _body1(a12a_ref, a12b_ref, f1_ref, x2a_ref, x2b_ref,
           w2s0_ref, w2n0_ref,
           b12_ref, h2_ref, g2r_ref):
    j = pl.program_id(0)
    for half, (aref, xref) in enumerate(((a12a_ref, x2a_ref),
                                         (a12b_ref, x2b_ref))):
        rows = pl.ds(half * HT + j * T, T)
        adj = aref[...]
        s2 = jnp.sum(jnp.where(adj > 0, f1_ref[...], 0.0),
                     axis=1, keepdims=True)
        au = adj.astype(jnp.uint32)
        b = jnp.zeros((T, NW), dtype=jnp.uint32)
        for k in range(PACK):
            b = b + (au[:, k * NW:(k + 1) * NW] << k)
        b12_ref[rows, :] = b

        c2 = jnp.sum(_bf(w2n0_ref[0, :]))
        a2 = jnp.sum(_bf(xref[...]) * _bf(w2s0_ref[0:1, :]),
                     axis=1, keepdims=True)
        g2 = jnp.maximum(a2 + _bf(s2) * c2, 0.0)
        g2r_ref[0:1, rows] = g2.reshape(1, T)

        r2n0 = jnp.sum(_bf(w2n0_ref[...]), axis=1)[None, :]
        h2 = jnp.maximum(_dott(xref[...], w2s0_ref[...]) + _bf(s2) * r2n0,
                         0.0)
        h2_ref[rows, :] = h2


def _body2(a21c_ref, a21d_ref, f2_ref, x1c_ref, x1d_ref,
           b12_ref, h2_ref, g2r_ref,
           w1s0_ref, w1n0_ref, w1s1_ref, w1n1_ref, w2s1_ref, w2n1_ref,
           o1a_ref, o1b_ref, o2_ref, s2p_scr):
    i = pl.program_id(0)

    @pl.when(i < P)
    def phase_b():
        j = i
        acc = jnp.zeros((N, 1), dtype=jnp.float32)
        for half, (aref, xref, oref) in enumerate(((a21c_ref, x1c_ref, o1a_ref),
                                                   (a21d_ref, x1d_ref, o1b_ref))):
            adj = aref[...]
            m = adj > 0
            mx0 = jnp.max(jnp.where(m, f2_ref[...], NEG), axis=1, keepdims=True)
            s1 = jnp.where(mx0 == NEG, 0.0, mx0)
            s1p = jnp.max(jnp.where(m, g2r_ref[...], 0.0),
                          axis=1, keepdims=True)

            c1 = jnp.sum(_bf(w1n0_ref[0, :]))
            a1 = jnp.sum(_bf(xref[...]) * _bf(w1s0_ref[0:1, :]),
                         axis=1, keepdims=True)
            g1 = jnp.maximum(a1 + _bf(s1) * c1, 0.0)

            r1n0 = jnp.sum(_bf(w1n0_ref[...]), axis=1)[None, :]
            r1n1 = jnp.sum(_bf(w1n1_ref[...]), axis=1)[None, :]
            h1 = jnp.maximum(_dott(xref[...], w1s0_ref[...]) + _bf(s1) * r1n0,
                             0.0)
            oref[...] = _dott(h1, w1s1_ref[...]) + _bf(s1p) * r1n1

            hi = g1.astype(jnp.bfloat16)
            lo = (g1 - hi.astype(jnp.float32)).astype(jnp.bfloat16)
            hl = jnp.concatenate([hi, lo], axis=1)
            base = QT * (half * P + j)
            bits = b12_ref[...]
            for q in range(QT):
                k = base + q
                mk = ((bits >> k) & jnp.uint32(1)).astype(jnp.bfloat16)
                seg = hl[q * NW:(q + 1) * NW, :]
                acc = acc + jnp.sum(_matmat(mk, seg), axis=1, keepdims=True)
        prev = jnp.where(j == 0, jnp.zeros((N, 1), jnp.float32), s2p_scr[...])
        s2p_scr[...] = prev + acc

    @pl.when(i >= P)
    def phase_c():
        j = i - P
        rows = pl.ds(j * (2 * T), 2 * T)
        h2 = h2_ref[rows, :]
        s2p = s2p_scr[rows, :]
        r2n1 = jnp.sum(_bf(w2n1_ref[...]), axis=1)[None, :]
        o2_ref[...] = _dott(h2, w2s1_ref[...]) + _bf(s2p) * r2n1


def kernel(x1, x2, adj_1to2, adj_2to1,
           l0_w1_self, l0_w1_neigh, l0_w2_self, l0_w2_neigh,
           l1_w1_self, l1_w1_neigh, l1_w2_self, l1_w2_neigh):
    f1 = x1[:, 0].reshape(1, N)
    f2 = x2[:, 0].reshape(1, N)

    top = lambda i: (jnp.minimum(i, P - 1), 0)
    bot = lambda i: (P + jnp.minimum(i, P - 1), 0)
    out_c = lambda i: (jnp.clip(i - P, 0, P - 1), 0)
    full = lambda i: (0, 0)

    b12, h2s, g2r = pl.pallas_call(
        _body1,
        grid=(P,),
        in_specs=[
            pl.BlockSpec((T, N), top),
            pl.BlockSpec((T, N), bot),
            pl.BlockSpec((1, N), full),
            pl.BlockSpec((T, D), top),
            pl.BlockSpec((T, D), bot),
            pl.BlockSpec((D, D), full),
            pl.BlockSpec((D, D), full),
        ],
        out_specs=[
            pl.BlockSpec((N, NW), full),
            pl.BlockSpec((N, D), full),
            pl.BlockSpec((1, N), full),
        ],
        out_shape=[
            jax.ShapeDtypeStruct((N, NW), jnp.uint32),
            jax.ShapeDtypeStruct((N, D), jnp.float32),
            jax.ShapeDtypeStruct((1, N), jnp.float32),
        ],
        compiler_params=pltpu.CompilerParams(
            dimension_semantics=("arbitrary",)),
    )(adj_1to2, adj_1to2, f1, x2, x2, l0_w2_self, l0_w2_neigh)

    o1a, o1b, o2 = pl.pallas_call(
        _body2,
        grid=(2 * P,),
        in_specs=[
            pl.BlockSpec((T, N), top),
            pl.BlockSpec((T, N), bot),
            pl.BlockSpec((1, N), full),
            pl.BlockSpec((T, D), top),
            pl.BlockSpec((T, D), bot),
            pl.BlockSpec((N, NW), full),
            pl.BlockSpec((N, D), full),
            pl.BlockSpec((1, N), full),
            pl.BlockSpec((D, D), full),
            pl.BlockSpec((D, D), full),
            pl.BlockSpec((D, D), full),
            pl.BlockSpec((D, D), full),
            pl.BlockSpec((D, D), full),
            pl.BlockSpec((D, D), full),
        ],
        out_specs=[
            pl.BlockSpec((T, D), top),
            pl.BlockSpec((T, D), top),
            pl.BlockSpec((2 * T, D), out_c),
        ],
        out_shape=[
            jax.ShapeDtypeStruct((HT, D), jnp.float32),
            jax.ShapeDtypeStruct((HT, D), jnp.float32),
            jax.ShapeDtypeStruct((N, D), jnp.float32),
        ],
        scratch_shapes=[
            pltpu.VMEM((N, 1), jnp.float32),
        ],
        compiler_params=pltpu.CompilerParams(
            dimension_semantics=("arbitrary",)),
    )(adj_2to1, adj_2to1, f2, x1, x1, b12, h2s, g2r,
      l0_w1_self, l0_w1_neigh, l1_w1_self, l1_w1_neigh,
      l1_w2_self, l1_w2_neigh)

    return (jnp.concatenate([o1a, o1b], axis=0), o2)
